# baseline (speedup 1.0000x reference)
import functools

import jax
import jax.numpy as jnp
from jax import lax
from jax.experimental import pallas as pl
from jax.experimental.pallas import tpu as pltpu

_THRESHOLD = 0.5
_NEG_POS_RATIO = 3
_ALPHA = 1.0


def _body(boxes_sm, labels_sm, scores_ref, locs_ref, priors_ref, out_ref,
          hard_ref, neg_scr, npos_scr, lab2d_scr, row_scr, ce2d_scr,
          *, P, C, O, R, B):
    i = pl.program_id(0)
    row_io = lax.broadcasted_iota(jnp.int32, (R, 128), 0)
    col_io = lax.broadcasted_iota(jnp.int32, (R, 128), 1)
    lin = row_io * 128 + col_io
    valid = lin < P

    pcx = priors_ref[0]
    pcy = priors_ref[1]
    pw = priors_ref[2]
    ph = priors_ref[3]
    px0 = pcx - pw * 0.5
    px1 = pcx + pw * 0.5
    py0 = pcy - ph * 0.5
    py1 = pcy + ph * 0.5
    parea = (px1 - px0) * (py1 - py0)

    best_ov = jnp.full((R, 128), -1.0, dtype=jnp.float32)
    best_o = jnp.zeros((R, 128), dtype=jnp.int32)
    pfeo = []
    for o in range(O):
        bcx = boxes_sm[i, 4 * o + 0]
        bcy = boxes_sm[i, 4 * o + 1]
        bw = boxes_sm[i, 4 * o + 2]
        bh = boxes_sm[i, 4 * o + 3]
        bx0 = bcx - bw * 0.5
        bx1 = bcx + bw * 0.5
        by0 = bcy - bh * 0.5
        by1 = bcy + bh * 0.5
        barea = (bx1 - bx0) * (by1 - by0)
        iw = jnp.clip(jnp.minimum(bx1, px1) - jnp.maximum(bx0, px0), 0.0, None)
        ih = jnp.clip(jnp.minimum(by1, py1) - jnp.maximum(by0, py0), 0.0, None)
        inter = iw * ih
        ov = inter / (barea + parea - inter)
        ov = jnp.where(valid, ov, -1.0)
        upd = ov > best_ov
        best_o = jnp.where(upd, o, best_o)
        best_ov = jnp.where(upd, ov, best_ov)
        m = jnp.max(ov)
        pfeo.append(jnp.min(jnp.where(ov == m, lin, P + 128)))

    forced = jnp.zeros((R, 128), dtype=jnp.bool_)
    ofep = best_o
    for o in range(O):
        mt = lin == pfeo[o]
        ofep = jnp.where(mt, o, ofep)
        forced = forced | mt
    ov_fep = jnp.where(forced, 1.0, best_ov)

    lab = jnp.zeros((R, 128), dtype=jnp.int32)
    bgcx = jnp.zeros((R, 128), dtype=jnp.float32)
    bgcy = jnp.zeros((R, 128), dtype=jnp.float32)
    bgw = jnp.ones((R, 128), dtype=jnp.float32)
    bgh = jnp.ones((R, 128), dtype=jnp.float32)
    for o in range(O):
        sel = ofep == o
        lab = jnp.where(sel, labels_sm[i, o], lab)
        bgcx = jnp.where(sel, boxes_sm[i, 4 * o + 0], bgcx)
        bgcy = jnp.where(sel, boxes_sm[i, 4 * o + 1], bgcy)
        bgw = jnp.where(sel, boxes_sm[i, 4 * o + 2], bgw)
        bgh = jnp.where(sel, boxes_sm[i, 4 * o + 3], bgh)
    lab = jnp.where(ov_fep < _THRESHOLD, 0, lab)
    pos = lab != 0
    posf = pos.astype(jnp.float32)
    n_pos = jnp.sum(posf)

    g_cx = (bgcx - pcx) / (pw * 0.1)
    g_cy = (bgcy - pcy) / (ph * 0.1)
    g_w = jnp.log(bgw / pw) * 5.0
    g_h = jnp.log(bgh / ph) * 5.0
    l = locs_ref[0]
    loc_term = (jnp.abs(l[0] - g_cx) + jnp.abs(l[1] - g_cy)
                + jnp.abs(l[2] - g_w) + jnp.abs(l[3] - g_h))
    loc_sum = jnp.sum(loc_term * posf)

    st = jnp.transpose(scores_ref[0], (1, 0))
    mx = jnp.max(st, axis=0, keepdims=True)
    es = jnp.sum(jnp.exp(st - mx), axis=0, keepdims=True)
    lse = mx + jnp.log(es)

    lab2d_scr[...] = lab
    for r in range(R):
        row_scr[0:1, pl.ds(r * 128, 128)] = lab2d_scr[r:r + 1, :]
    lab_row = row_scr[0:1, 0:P]
    cls = lax.broadcasted_iota(jnp.int32, (C, P), 0)
    tgt = jnp.sum(jnp.where(cls == lab_row, st, 0.0), axis=0, keepdims=True)
    ce_row = lse - tgt

    ce2d_scr[0:1, 0:P] = ce_row
    pad_n = R * 128 - P
    if pad_n:
        ce2d_scr[0:1, pl.ds(P, pad_n)] = jnp.zeros((1, pad_n), jnp.float32)
    for r in range(R):
        lab2d_scr[r:r + 1, :] = lax.bitcast_convert_type(
            ce2d_scr[0:1, pl.ds(r * 128, 128)], jnp.int32)
    ce = lax.bitcast_convert_type(lab2d_scr[...], jnp.float32)
    conf_pos = jnp.sum(ce * posf)
    neg_ce = jnp.where(pos | jnp.logical_not(valid), 0.0, ce)

    neg_scr[i] = neg_ce
    npos_scr[i] = jnp.full((1, 128), n_pos, dtype=jnp.float32)

    io = lax.broadcasted_iota(jnp.int32, (1, 1, 128), 2)
    row = (jnp.where(io == 0, n_pos, 0.0) + jnp.where(io == 1, loc_sum, 0.0)
           + jnp.where(io == 2, conf_pos, 0.0))
    out_ref[...] = row

    @pl.when(i == B - 1)
    def _mine():
        neg_all = neg_scr[...]
        bits = lax.bitcast_convert_type(neg_all, jnp.int32)
        k3 = _NEG_POS_RATIO * npos_scr[...][:, :, 0:1]

        def bs_step(j, t):
            t2 = t | jnp.left_shift(jnp.int32(1), 30 - j)
            sel = jnp.where(bits >= t2, 1.0, 0.0)
            cnt = jnp.sum(jnp.sum(sel, axis=1, keepdims=True), axis=2,
                          keepdims=True)
            return jnp.where(cnt >= k3, t2, t)

        t = lax.fori_loop(0, 31, bs_step,
                          jnp.zeros((B, 1, 1), dtype=jnp.int32))
        gt = jnp.where(bits > t, 1.0, 0.0)
        sum_gt = jnp.sum(jnp.sum(neg_all * gt, axis=1, keepdims=True),
                         axis=2, keepdims=True)
        cnt_gt = jnp.sum(jnp.sum(gt, axis=1, keepdims=True), axis=2,
                         keepdims=True)
        eqv = jnp.where(bits == t, neg_all, 0.0)
        tval = jnp.max(jnp.max(eqv, axis=1, keepdims=True), axis=2,
                       keepdims=True)
        hard_img = sum_gt + (k3 - cnt_gt) * tval
        hard_t = jnp.sum(hard_img)
        io2 = lax.broadcasted_iota(jnp.int32, (1, 1, 128), 2)
        hard_ref[...] = jnp.where(io2 == 0, hard_t, 0.0)


def kernel(predicted_locs, predicted_scores, boxes, labels, priors_cxcy):
    B, P, C = predicted_scores.shape
    O = boxes.shape[1]
    R = (P + 127) // 128
    pad = R * 128 - P

    locs_r = jnp.moveaxis(predicted_locs, 2, 1)
    locs_r = jnp.pad(locs_r, ((0, 0), (0, 0), (0, pad))).reshape(B, 4, R, 128)
    priors_r = jnp.pad(priors_cxcy.T, ((0, 0), (0, pad)),
                       constant_values=1.0).reshape(4, R, 128)
    boxes_r = boxes.reshape(B, 4 * O)
    labels_r = labels.astype(jnp.int32)

    grid_spec = pltpu.PrefetchScalarGridSpec(
        num_scalar_prefetch=2,
        grid=(B,),
        in_specs=[
            pl.BlockSpec((1, P, C), lambda i, *_: (i, 0, 0)),
            pl.BlockSpec((1, 4, R, 128), lambda i, *_: (i, 0, 0, 0)),
            pl.BlockSpec((4, R, 128), lambda i, *_: (0, 0, 0)),
        ],
        out_specs=[
            pl.BlockSpec((1, 1, 128), lambda i, *_: (i, 0, 0)),
            pl.BlockSpec((1, 1, 128), lambda i, *_: (0, 0, 0)),
        ],
        scratch_shapes=[
            pltpu.VMEM((B, R, 128), jnp.float32),
            pltpu.VMEM((B, 1, 128), jnp.float32),
            pltpu.VMEM((R, 128), jnp.int32),
            pltpu.VMEM((1, R * 128), jnp.int32),
            pltpu.VMEM((1, R * 128), jnp.float32),
        ],
    )
    partials, hard_row = pl.pallas_call(
        functools.partial(_body, P=P, C=C, O=O, R=R, B=B),
        grid_spec=grid_spec,
        out_shape=[
            jax.ShapeDtypeStruct((B, 1, 128), jnp.float32),
            jax.ShapeDtypeStruct((1, 1, 128), jnp.float32),
        ],
    )(boxes_r, labels_r, predicted_scores, locs_r, priors_r)

    n_pos_total = jnp.sum(partials[:, 0, 0])
    loc_sum_t = jnp.sum(partials[:, 0, 1])
    conf_pos_t = jnp.sum(partials[:, 0, 2])
    hard_t = hard_row[0, 0, 0]
    loc_loss = loc_sum_t / (n_pos_total * 4.0)
    conf_loss = (hard_t + conf_pos_t) / n_pos_total
    total = conf_loss + _ALPHA * loc_loss
    return (conf_loss, loc_loss, total)


# speedup vs baseline: 1.5865x; 1.5865x over previous
import functools

import jax
import jax.numpy as jnp
from jax import lax
from jax.experimental import pallas as pl
from jax.experimental.pallas import tpu as pltpu

_THRESHOLD = 0.5
_NEG_POS_RATIO = 3
_ALPHA = 1.0


def _body(boxes_sm, labels_sm, scores_ref, locs_ref, priors_ref, out_ref,
          hard_ref, neg_scr, npos_scr, *, P, C, O, R, B):
    i = pl.program_id(0)
    row_io = lax.broadcasted_iota(jnp.int32, (R, 128), 0)
    col_io = lax.broadcasted_iota(jnp.int32, (R, 128), 1)
    lin = row_io * 128 + col_io
    valid = lin < P

    pcx = priors_ref[0]
    pcy = priors_ref[1]
    pw = priors_ref[2]
    ph = priors_ref[3]
    px0 = pcx - pw * 0.5
    px1 = pcx + pw * 0.5
    py0 = pcy - ph * 0.5
    py1 = pcy + ph * 0.5
    parea = (px1 - px0) * (py1 - py0)

    ovs = []
    for o in range(O):
        bcx = boxes_sm[i, 4 * o + 0]
        bcy = boxes_sm[i, 4 * o + 1]
        bw = boxes_sm[i, 4 * o + 2]
        bh = boxes_sm[i, 4 * o + 3]
        bx0 = bcx - bw * 0.5
        bx1 = bcx + bw * 0.5
        by0 = bcy - bh * 0.5
        by1 = bcy + bh * 0.5
        barea = (bx1 - bx0) * (by1 - by0)
        iw = jnp.clip(jnp.minimum(bx1, px1) - jnp.maximum(bx0, px0), 0.0, None)
        ih = jnp.clip(jnp.minimum(by1, py1) - jnp.maximum(by0, py0), 0.0, None)
        inter = iw * ih
        ov = inter / (barea + parea - inter)
        ovs.append(jnp.where(valid, ov, -1.0))

    nodes = [(ov, jnp.full((R, 128), o, dtype=jnp.int32))
             for o, ov in enumerate(ovs)]
    while len(nodes) > 1:
        nxt = []
        for a in range(0, len(nodes), 2):
            (va, oa), (vb, ob) = nodes[a], nodes[a + 1]
            take_b = vb > va
            nxt.append((jnp.where(take_b, vb, va), jnp.where(take_b, ob, oa)))
        nodes = nxt
    best_ov, best_o = nodes[0]

    ov_all = jnp.concatenate([ov[None] for ov in ovs], axis=0)
    m8 = jnp.max(jnp.max(ov_all, axis=1, keepdims=True), axis=2, keepdims=True)
    idx_src = jnp.where(ov_all == m8, lin[None], P + 128)
    idx8 = jnp.min(jnp.min(idx_src, axis=1, keepdims=True), axis=2,
                   keepdims=True)

    mt_all = lin[None] == idx8
    oid8 = lax.broadcasted_iota(jnp.int32, (O, 1, 1), 0)
    win = jnp.max(jnp.where(mt_all, oid8, -1), axis=0)
    forced = win >= 0
    ofep = jnp.where(forced, win, best_o)
    ov_fep = jnp.where(forced, 1.0, best_ov)

    gn = [(ofep == o,
           jnp.full((R, 128), labels_sm[i, o], dtype=jnp.int32),
           boxes_sm[i, 4 * o + 0], boxes_sm[i, 4 * o + 1],
           boxes_sm[i, 4 * o + 2], boxes_sm[i, 4 * o + 3])
          for o in range(O)]
    while len(gn) > 1:
        nxt = []
        for a in range(0, len(gn), 2):
            na, nb = gn[a], gn[a + 1]
            sa = na[0]
            merged = [sa | nb[0]]
            for va, vb in zip(na[1:], nb[1:]):
                merged.append(jnp.where(sa, va, vb))
            nxt.append(tuple(merged))
        gn = nxt
    _, lab, bgcx, bgcy, bgw, bgh = gn[0]
    lab = jnp.where(ov_fep < _THRESHOLD, 0, lab)
    pos = lab != 0
    posf = pos.astype(jnp.float32)
    n_pos = jnp.sum(posf)

    g_cx = (bgcx - pcx) / (pw * 0.1)
    g_cy = (bgcy - pcy) / (ph * 0.1)
    g_w = jnp.log(bgw / pw) * 5.0
    g_h = jnp.log(bgh / ph) * 5.0
    l = locs_ref[0]
    loc_term = (jnp.abs(l[0] - g_cx) + jnp.abs(l[1] - g_cy)
                + jnp.abs(l[2] - g_w) + jnp.abs(l[3] - g_h))
    loc_sum = jnp.sum(loc_term * posf)

    s = scores_ref[0]
    mx = jnp.max(s, axis=0)
    es = jnp.sum(jnp.exp(s - mx[None]), axis=0)
    lse = mx + jnp.log(es)
    tgt = jnp.zeros((R, 128), dtype=jnp.float32)
    for c in range(C):
        tgt = jnp.where(lab == c, s[c], tgt)
    ce = lse - tgt
    conf_pos = jnp.sum(ce * posf)
    neg_ce = jnp.where(pos | jnp.logical_not(valid), 0.0, ce)

    neg_scr[i] = neg_ce
    npos_scr[i] = jnp.full((1, 128), n_pos, dtype=jnp.float32)

    io = lax.broadcasted_iota(jnp.int32, (1, 1, 128), 2)
    row = (jnp.where(io == 0, n_pos, 0.0) + jnp.where(io == 1, loc_sum, 0.0)
           + jnp.where(io == 2, conf_pos, 0.0))
    out_ref[...] = row

    @pl.when(i == B - 1)
    def _mine():
        neg_all = neg_scr[...]
        bits = lax.bitcast_convert_type(neg_all, jnp.int32)
        k3 = _NEG_POS_RATIO * npos_scr[...][:, :, 0:1]

        def bs_step(j, t):
            t2 = t | jnp.left_shift(jnp.int32(1), 30 - j)
            sel = jnp.where(bits >= t2, 1.0, 0.0)
            cnt = jnp.sum(jnp.sum(sel, axis=1, keepdims=True), axis=2,
                          keepdims=True)
            return jnp.where(cnt >= k3, t2, t)

        t = lax.fori_loop(0, 31, bs_step,
                          jnp.zeros((B, 1, 1), dtype=jnp.int32))
        gt = jnp.where(bits > t, 1.0, 0.0)
        sum_gt = jnp.sum(jnp.sum(neg_all * gt, axis=1, keepdims=True),
                         axis=2, keepdims=True)
        cnt_gt = jnp.sum(jnp.sum(gt, axis=1, keepdims=True), axis=2,
                         keepdims=True)
        eqv = jnp.where(bits == t, neg_all, 0.0)
        tval = jnp.max(jnp.max(eqv, axis=1, keepdims=True), axis=2,
                       keepdims=True)
        hard_img = sum_gt + (k3 - cnt_gt) * tval
        hard_t = jnp.sum(hard_img)
        io2 = lax.broadcasted_iota(jnp.int32, (1, 1, 128), 2)
        hard_ref[...] = jnp.where(io2 == 0, hard_t, 0.0)


def kernel(predicted_locs, predicted_scores, boxes, labels, priors_cxcy):
    B, P, C = predicted_scores.shape
    O = boxes.shape[1]
    R = (P + 127) // 128
    pad = R * 128 - P

    scores_r = jnp.moveaxis(predicted_scores, 2, 1)
    scores_r = jnp.pad(scores_r, ((0, 0), (0, 0), (0, pad))).reshape(B, C, R, 128)
    locs_r = jnp.moveaxis(predicted_locs, 2, 1)
    locs_r = jnp.pad(locs_r, ((0, 0), (0, 0), (0, pad))).reshape(B, 4, R, 128)
    priors_r = jnp.pad(priors_cxcy.T, ((0, 0), (0, pad)),
                       constant_values=1.0).reshape(4, R, 128)
    boxes_r = boxes.reshape(B, 4 * O)
    labels_r = labels.astype(jnp.int32)

    grid_spec = pltpu.PrefetchScalarGridSpec(
        num_scalar_prefetch=2,
        grid=(B,),
        in_specs=[
            pl.BlockSpec((1, C, R, 128), lambda i, *_: (i, 0, 0, 0)),
            pl.BlockSpec((1, 4, R, 128), lambda i, *_: (i, 0, 0, 0)),
            pl.BlockSpec((4, R, 128), lambda i, *_: (0, 0, 0)),
        ],
        out_specs=[
            pl.BlockSpec((1, 1, 128), lambda i, *_: (i, 0, 0)),
            pl.BlockSpec((1, 1, 128), lambda i, *_: (0, 0, 0)),
        ],
        scratch_shapes=[
            pltpu.VMEM((B, R, 128), jnp.float32),
            pltpu.VMEM((B, 1, 128), jnp.float32),
        ],
    )
    partials, hard_row = pl.pallas_call(
        functools.partial(_body, P=P, C=C, O=O, R=R, B=B),
        grid_spec=grid_spec,
        out_shape=[
            jax.ShapeDtypeStruct((B, 1, 128), jnp.float32),
            jax.ShapeDtypeStruct((1, 1, 128), jnp.float32),
        ],
    )(boxes_r, labels_r, scores_r, locs_r, priors_r)

    n_pos_total = jnp.sum(partials[:, 0, 0])
    loc_sum_t = jnp.sum(partials[:, 0, 1])
    conf_pos_t = jnp.sum(partials[:, 0, 2])
    hard_t = hard_row[0, 0, 0]
    loc_loss = loc_sum_t / (n_pos_total * 4.0)
    conf_loss = (hard_t + conf_pos_t) / n_pos_total
    total = conf_loss + _ALPHA * loc_loss
    return (conf_loss, loc_loss, total)


# speedup vs baseline: 1.7802x; 1.1221x over previous
import functools

import jax
import jax.numpy as jnp
from jax import lax
from jax.experimental import pallas as pl
from jax.experimental.pallas import tpu as pltpu

_THRESHOLD = 0.5
_NEG_POS_RATIO = 3
_ALPHA = 1.0


def _body(boxes_sm, labels_sm, scores_ref, locs_ref, priors_ref, out_ref,
          hard_ref, neg_scr, npos_scr, lab2d_scr, labrow_scr, cerow_scr,
          cetile_scr, *, P, C, O, R, B):
    i = pl.program_id(0)
    row_io = lax.broadcasted_iota(jnp.int32, (R, 128), 0)
    col_io = lax.broadcasted_iota(jnp.int32, (R, 128), 1)
    lin = row_io * 128 + col_io
    valid = lin < P

    pcx = priors_ref[0]
    pcy = priors_ref[1]
    pw = priors_ref[2]
    ph = priors_ref[3]
    px0 = pcx - pw * 0.5
    px1 = pcx + pw * 0.5
    py0 = pcy - ph * 0.5
    py1 = pcy + ph * 0.5
    parea = (px1 - px0) * (py1 - py0)

    ovs = []
    for o in range(O):
        bcx = boxes_sm[i, 4 * o + 0]
        bcy = boxes_sm[i, 4 * o + 1]
        bw = boxes_sm[i, 4 * o + 2]
        bh = boxes_sm[i, 4 * o + 3]
        bx0 = bcx - bw * 0.5
        bx1 = bcx + bw * 0.5
        by0 = bcy - bh * 0.5
        by1 = bcy + bh * 0.5
        barea = (bx1 - bx0) * (by1 - by0)
        iw = jnp.clip(jnp.minimum(bx1, px1) - jnp.maximum(bx0, px0), 0.0, None)
        ih = jnp.clip(jnp.minimum(by1, py1) - jnp.maximum(by0, py0), 0.0, None)
        inter = iw * ih
        ov = inter / (barea + parea - inter)
        ovs.append(jnp.where(valid, ov, -1.0))

    nodes = [(ov, jnp.full((R, 128), o, dtype=jnp.int32))
             for o, ov in enumerate(ovs)]
    while len(nodes) > 1:
        nxt = []
        for a in range(0, len(nodes), 2):
            (va, oa), (vb, ob) = nodes[a], nodes[a + 1]
            take_b = vb > va
            nxt.append((jnp.where(take_b, vb, va), jnp.where(take_b, ob, oa)))
        nodes = nxt
    best_ov, best_o = nodes[0]

    ov_all = jnp.concatenate([ov[None] for ov in ovs], axis=0)
    m8 = jnp.max(jnp.max(ov_all, axis=1, keepdims=True), axis=2, keepdims=True)
    idx_src = jnp.where(ov_all == m8, lin[None], P + 128)
    idx8 = jnp.min(jnp.min(idx_src, axis=1, keepdims=True), axis=2,
                   keepdims=True)

    mt_all = lin[None] == idx8
    oid8 = lax.broadcasted_iota(jnp.int32, (O, 1, 1), 0)
    win = jnp.max(jnp.where(mt_all, oid8, -1), axis=0)
    forced = win >= 0
    ofep = jnp.where(forced, win, best_o)
    ov_fep = jnp.where(forced, 1.0, best_ov)

    gn = [(ofep == o,
           jnp.full((R, 128), labels_sm[i, o], dtype=jnp.int32),
           boxes_sm[i, 4 * o + 0], boxes_sm[i, 4 * o + 1],
           boxes_sm[i, 4 * o + 2], boxes_sm[i, 4 * o + 3])
          for o in range(O)]
    while len(gn) > 1:
        nxt = []
        for a in range(0, len(gn), 2):
            na, nb = gn[a], gn[a + 1]
            sa = na[0]
            merged = [sa | nb[0]]
            for va, vb in zip(na[1:], nb[1:]):
                merged.append(jnp.where(sa, va, vb))
            nxt.append(tuple(merged))
        gn = nxt
    _, lab, bgcx, bgcy, bgw, bgh = gn[0]
    lab = jnp.where(ov_fep < _THRESHOLD, 0, lab)
    pos = lab != 0
    posf = pos.astype(jnp.float32)
    n_pos = jnp.sum(posf)

    g_cx = (bgcx - pcx) / (pw * 0.1)
    g_cy = (bgcy - pcy) / (ph * 0.1)
    g_w = jnp.log(bgw / pw) * 5.0
    g_h = jnp.log(bgh / ph) * 5.0
    l = locs_ref[0]
    loc_term = (jnp.abs(l[0] - g_cx) + jnp.abs(l[1] - g_cy)
                + jnp.abs(l[2] - g_w) + jnp.abs(l[3] - g_h))
    loc_sum = jnp.sum(loc_term * posf)

    st = scores_ref[0]
    mx = jnp.max(st, axis=0, keepdims=True)
    es = jnp.sum(jnp.exp(st - mx), axis=0, keepdims=True)
    lse = mx + jnp.log(es)

    lab2d_scr[...] = lab
    for r in range(R):
        labrow_scr[0:1, pl.ds(r * 128, 128)] = lab2d_scr[r:r + 1, :]
    lab_row = labrow_scr[0:1, 0:P]
    cls = lax.broadcasted_iota(jnp.int32, (C, P), 0)
    tgt = jnp.sum(jnp.where(cls == lab_row, st, 0.0), axis=0, keepdims=True)
    ce_row = lse - tgt

    cerow_scr[0:1, 0:P] = ce_row
    pad_n = R * 128 - P
    if pad_n:
        cerow_scr[0:1, pl.ds(P, pad_n)] = jnp.zeros((1, pad_n), jnp.float32)
    for r in range(R):
        cetile_scr[r:r + 1, :] = cerow_scr[0:1, pl.ds(r * 128, 128)]
    ce = cetile_scr[...]
    conf_pos = jnp.sum(ce * posf)
    neg_ce = jnp.where(pos | jnp.logical_not(valid), 0.0, ce)

    neg_scr[i] = neg_ce
    npos_scr[i] = jnp.full((1, 128), n_pos, dtype=jnp.float32)

    io = lax.broadcasted_iota(jnp.int32, (1, 1, 128), 2)
    row = (jnp.where(io == 0, n_pos, 0.0) + jnp.where(io == 1, loc_sum, 0.0)
           + jnp.where(io == 2, conf_pos, 0.0))
    out_ref[...] = row

    @pl.when(i == B - 1)
    def _mine():
        neg_all = neg_scr[...]
        bits = lax.bitcast_convert_type(neg_all, jnp.int32)
        k3 = _NEG_POS_RATIO * npos_scr[...][:, :, 0:1]

        def bs_step(j, t):
            t2 = t | jnp.left_shift(jnp.int32(1), 30 - j)
            sel = jnp.where(bits >= t2, 1.0, 0.0)
            cnt = jnp.sum(jnp.sum(sel, axis=1, keepdims=True), axis=2,
                          keepdims=True)
            return jnp.where(cnt >= k3, t2, t)

        t = lax.fori_loop(0, 31, bs_step,
                          jnp.zeros((B, 1, 1), dtype=jnp.int32))
        gt = jnp.where(bits > t, 1.0, 0.0)
        sum_gt = jnp.sum(jnp.sum(neg_all * gt, axis=1, keepdims=True),
                         axis=2, keepdims=True)
        cnt_gt = jnp.sum(jnp.sum(gt, axis=1, keepdims=True), axis=2,
                         keepdims=True)
        eqv = jnp.where(bits == t, neg_all, 0.0)
        tval = jnp.max(jnp.max(eqv, axis=1, keepdims=True), axis=2,
                       keepdims=True)
        hard_img = sum_gt + (k3 - cnt_gt) * tval
        hard_t = jnp.sum(hard_img)
        io2 = lax.broadcasted_iota(jnp.int32, (1, 1, 128), 2)
        hard_ref[...] = jnp.where(io2 == 0, hard_t, 0.0)


def kernel(predicted_locs, predicted_scores, boxes, labels, priors_cxcy):
    B, P, C = predicted_scores.shape
    O = boxes.shape[1]
    R = (P + 127) // 128
    pad = R * 128 - P

    scores_r = jnp.moveaxis(predicted_scores, 2, 1)
    locs_r = jnp.moveaxis(predicted_locs, 2, 1)
    locs_r = jnp.pad(locs_r, ((0, 0), (0, 0), (0, pad))).reshape(B, 4, R, 128)
    priors_r = jnp.pad(priors_cxcy.T, ((0, 0), (0, pad)),
                       constant_values=1.0).reshape(4, R, 128)
    boxes_r = boxes.reshape(B, 4 * O)
    labels_r = labels.astype(jnp.int32)

    grid_spec = pltpu.PrefetchScalarGridSpec(
        num_scalar_prefetch=2,
        grid=(B,),
        in_specs=[
            pl.BlockSpec((1, C, P), lambda i, *_: (i, 0, 0)),
            pl.BlockSpec((1, 4, R, 128), lambda i, *_: (i, 0, 0, 0)),
            pl.BlockSpec((4, R, 128), lambda i, *_: (0, 0, 0)),
        ],
        out_specs=[
            pl.BlockSpec((1, 1, 128), lambda i, *_: (i, 0, 0)),
            pl.BlockSpec((1, 1, 128), lambda i, *_: (0, 0, 0)),
        ],
        scratch_shapes=[
            pltpu.VMEM((B, R, 128), jnp.float32),
            pltpu.VMEM((B, 1, 128), jnp.float32),
            pltpu.VMEM((R, 128), jnp.int32),
            pltpu.VMEM((1, R * 128), jnp.int32),
            pltpu.VMEM((1, R * 128), jnp.float32),
            pltpu.VMEM((R, 128), jnp.float32),
        ],
    )
    partials, hard_row = pl.pallas_call(
        functools.partial(_body, P=P, C=C, O=O, R=R, B=B),
        grid_spec=grid_spec,
        out_shape=[
            jax.ShapeDtypeStruct((B, 1, 128), jnp.float32),
            jax.ShapeDtypeStruct((1, 1, 128), jnp.float32),
        ],
    )(boxes_r, labels_r, scores_r, locs_r, priors_r)

    n_pos_total = jnp.sum(partials[:, 0, 0])
    loc_sum_t = jnp.sum(partials[:, 0, 1])
    conf_pos_t = jnp.sum(partials[:, 0, 2])
    hard_t = hard_row[0, 0, 0]
    loc_loss = loc_sum_t / (n_pos_total * 4.0)
    conf_loss = (hard_t + conf_pos_t) / n_pos_total
    total = conf_loss + _ALPHA * loc_loss
    return (conf_loss, loc_loss, total)
